# baseline (speedup 1.0000x reference)
import functools

import jax
import jax.numpy as jnp
from jax import lax
from jax.experimental import pallas as pl
from jax.experimental.pallas import tpu as pltpu
from jax.experimental.pallas import tpu_sc as plsc

N = 25000
ND = 2 * N
E = 500000
EP = 524288
F = 128
H = 4
D = 32
NC = 2
NS = 16
NW = NC * NS
ACC = 50176

_SQRT_D = 5.656854249492381
_F32_MAX = 3.4028235e38


def _gelu(x):
    return x * 0.5 * (1.0 + lax.erf(x * 0.7071067811865476))


def _proj_body(x_ref, w_ref, b_ref, q_ref, k_ref, v_ref):
    y = jnp.dot(x_ref[...], w_ref[...], preferred_element_type=jnp.float32)
    y = y + b_ref[...]
    q_ref[...] = y[:, 0:F]
    k_ref[...] = y[:, F:2 * F]
    v_ref[...] = y[:, 2 * F:3 * F]


def _proj(x, w, b):
    bm = 1000
    grid = N // bm
    return pl.pallas_call(
        _proj_body,
        grid=(grid,),
        in_specs=[
            pl.BlockSpec((bm, F), lambda i: (i, 0)),
            pl.BlockSpec((F, 3 * F), lambda i: (0, 0)),
            pl.BlockSpec((1, 3 * F), lambda i: (0, 0)),
        ],
        out_specs=[pl.BlockSpec((bm, F), lambda i: (i, 0))] * 3,
        out_shape=[jax.ShapeDtypeStruct((N, F), jnp.float32)] * 3,
    )(x, w, b)


_EPW = EP // NW
_GC = 128
_NB = 2
_GN = _EPW // (_GC * _NB)


def _sc_gather_body(qt, kvt, dstp, srcp, qo, kvo,
                    idxd, idxs, qr, kvr, semg, semw):
    c = lax.axis_index("c")
    s = lax.axis_index("s")
    wid = s * NC + c
    base = wid * _EPW

    def body(g, carry):
        off0 = base + g * _NB * _GC
        pltpu.sync_copy(dstp.at[pl.ds(off0, _NB * _GC)], idxd)
        pltpu.sync_copy(srcp.at[pl.ds(off0, _NB * _GC)], idxs)
        handles = []
        for b in range(_NB):
            @pl.when(g > 0)
            def _():
                pltpu.make_async_copy(qr[b], qo.at[pl.ds(off0, _GC)],
                                      semw[2 * b]).wait()
                pltpu.make_async_copy(kvr[b], kvo.at[pl.ds(off0, _GC)],
                                      semw[2 * b + 1]).wait()

            handles.append((
                pltpu.async_copy(qt.at[idxd.at[pl.ds(b * _GC, _GC)]],
                                 qr[b], semg[2 * b]),
                pltpu.async_copy(kvt.at[idxs.at[pl.ds(b * _GC, _GC)]],
                                 kvr[b], semg[2 * b + 1]),
            ))
        for b in range(_NB):
            off = off0 + b * _GC
            hq, hkv = handles[b]
            hq.wait()
            pltpu.async_copy(qr[b], qo.at[pl.ds(off, _GC)], semw[2 * b])
            hkv.wait()
            pltpu.async_copy(kvr[b], kvo.at[pl.ds(off, _GC)], semw[2 * b + 1])
        return carry

    lax.fori_loop(0, _GN, body, 0)
    for b in range(_NB):
        pltpu.make_async_copy(qr[b], qo.at[pl.ds(0, _GC)], semw[2 * b]).wait()
        pltpu.make_async_copy(kvr[b], kvo.at[pl.ds(0, _GC)],
                              semw[2 * b + 1]).wait()


def _sc_gather(q_cat, kv_cat, dstp, srcp):
    f = pl.kernel(
        _sc_gather_body,
        mesh=plsc.VectorSubcoreMesh(core_axis_name="c", subcore_axis_name="s"),
        out_type=[jax.ShapeDtypeStruct((EP, F), jnp.float32),
                  jax.ShapeDtypeStruct((EP, 2 * F), jnp.float32)],
        scratch_types=[
            pltpu.VMEM((_NB * _GC,), jnp.int32),
            pltpu.VMEM((_NB * _GC,), jnp.int32),
            [pltpu.VMEM((_GC, F), jnp.float32) for _ in range(_NB)],
            [pltpu.VMEM((_GC, 2 * F), jnp.float32) for _ in range(_NB)],
            [pltpu.SemaphoreType.DMA for _ in range(2 * _NB)],
            [pltpu.SemaphoreType.DMA for _ in range(2 * _NB)],
        ],
        compiler_params=pltpu.CompilerParams(use_tc_tiling_on_sc=False),
    )
    return f(q_cat, kv_cat, dstp, srcp)


_BE = 2048


def _edge_body(q_ref, kv_ref, m_ref):
    i = pl.program_id(0)
    q = q_ref[...]
    kv = kv_ref[...]
    k = kv[:, 0:F]
    v = kv[:, F:2 * F]
    rows = i * _BE + lax.broadcasted_iota(jnp.int32, (_BE, 1), 0)
    valid = rows < E
    es = []
    for h in range(H):
        sl = slice(h * D, (h + 1) * D)
        s = jnp.sum(q[:, sl] * k[:, sl], axis=1, keepdims=True)
        e = jnp.where(valid, jnp.exp(s), 0.0)
        es.append(e)
        m_ref[h] = v[:, sl] * e
    m_ref[H] = jnp.concatenate(es + [jnp.zeros((_BE, D - H), jnp.float32)], axis=1)


def _edge_math(Q, KV):
    grid = EP // _BE
    return pl.pallas_call(
        _edge_body,
        grid=(grid,),
        in_specs=[pl.BlockSpec((_BE, F), lambda i: (i, 0)),
                  pl.BlockSpec((_BE, 2 * F), lambda i: (i, 0))],
        out_specs=pl.BlockSpec((H + 1, _BE, D), lambda i: (0, i, 0)),
        out_shape=jax.ShapeDtypeStruct((H + 1, EP, D), jnp.float32),
    )(Q, KV)


_EPC = EP // NC
_EPT = _EPC // NS
_SCK = 512
_SN = _EPT // _SCK
_ZR = ACC // NS


def _sc_scatter_body(m, dstp, po, vals, idx, acc):
    c = lax.axis_index("c")
    s = lax.axis_index("s")

    for h in range(H + 1):
        def zb(i, carry):
            vals[i, pl.ds(0, 16)] = jnp.zeros((16,), jnp.float32)
            vals[i, pl.ds(16, 16)] = jnp.zeros((16,), jnp.float32)
            return carry

        lax.fori_loop(0, _SCK, zb, 0)
        for j in range(6):
            pltpu.sync_copy(vals, acc.at[pl.ds(s * _ZR + j * _SCK, _SCK)])
        pltpu.sync_copy(vals.at[pl.ds(0, 64)],
                        acc.at[pl.ds(s * _ZR + 6 * _SCK, 64)])
        plsc.subcore_barrier()

        base = c * _EPC + s * _EPT

        def body(g, carry):
            off = base + g * _SCK
            pltpu.sync_copy(dstp.at[pl.ds(off, _SCK)], idx)
            pltpu.sync_copy(m.at[h, pl.ds(off, _SCK)], vals)
            pltpu.sync_copy(vals, acc.at[idx], add=True)
            return carry

        lax.fori_loop(0, _SN, body, 0)
        plsc.subcore_barrier()

        for j in range(6):
            r0 = s * _ZR + j * _SCK
            pltpu.sync_copy(acc.at[pl.ds(r0, _SCK)], po.at[c, h, pl.ds(r0, _SCK)])
        r0 = s * _ZR + 6 * _SCK
        pltpu.sync_copy(acc.at[pl.ds(r0, 64)], po.at[c, h, pl.ds(r0, 64)])
        plsc.subcore_barrier()


def _sc_scatter(M, dstp):
    f = pl.kernel(
        _sc_scatter_body,
        mesh=plsc.VectorSubcoreMesh(core_axis_name="c", subcore_axis_name="s"),
        out_type=jax.ShapeDtypeStruct((NC, H + 1, ACC, D), jnp.float32),
        scratch_types=[
            pltpu.VMEM((_SCK, D), jnp.float32),
            pltpu.VMEM((_SCK,), jnp.int32),
            pltpu.VMEM_SHARED((ACC, D), jnp.float32),
        ],
        compiler_params=pltpu.CompilerParams(use_tc_tiling_on_sc=False),
    )
    return f(M, dstp)


def _post_body(p_ref, x_ref, w_ref, b_ref, sk_ref, g_ref, bn_ref, o_ref,
               *, mode):
    p = p_ref[...]
    ps = p[0] + p[1]
    cols = []
    for h in range(H):
        cols.append(ps[h] / (ps[H][:, h:h + 1] + 1e-16))
    agg = jnp.concatenate(cols, axis=1)
    a = jnp.dot(_gelu(agg), w_ref[...], preferred_element_type=jnp.float32)
    a = a + b_ref[...]
    sk = jax.nn.sigmoid(sk_ref[0, 0])
    a = sk * a + (1.0 - sk) * x_ref[...]
    if mode == "ln_gelu":
        mu = jnp.mean(a, axis=-1, keepdims=True)
        var = jnp.mean((a - mu) * (a - mu), axis=-1, keepdims=True)
        a = (a - mu) / jnp.sqrt(var + 1e-5) * g_ref[...] + bn_ref[...]
    o_ref[...] = _gelu(a)


def _post(P_nt, x_nt, w_out, b_out, skip, g, bn, mode):
    bm = 1000
    grid = N // bm
    body = functools.partial(_post_body, mode=mode)
    return pl.pallas_call(
        body,
        grid=(grid,),
        in_specs=[
            pl.BlockSpec((NC, H + 1, bm, D), lambda i: (0, 0, i, 0)),
            pl.BlockSpec((bm, F), lambda i: (i, 0)),
            pl.BlockSpec((F, F), lambda i: (0, 0)),
            pl.BlockSpec((1, F), lambda i: (0, 0)),
            pl.BlockSpec((1, 1), lambda i: (0, 0)),
            pl.BlockSpec((1, F), lambda i: (0, 0)),
            pl.BlockSpec((1, F), lambda i: (0, 0)),
        ],
        out_specs=pl.BlockSpec((bm, F), lambda i: (i, 0)),
        out_shape=jax.ShapeDtypeStruct((N, F), jnp.float32),
    )(P_nt, x_nt, w_out, b_out, skip, g, bn)


def _colsum_body(x_ref, w_ref, b_ref, o_ref):
    i = pl.program_id(0)
    a = jnp.dot(x_ref[...], w_ref[...], preferred_element_type=jnp.float32)
    a = _gelu(a + b_ref[...])
    s = jnp.sum(a, axis=0, keepdims=True)

    @pl.when(i == 0)
    def _():
        o_ref[...] = s

    @pl.when(i > 0)
    def _():
        o_ref[...] += s


def _colsum(x, w, b):
    bm = 1000
    grid = N // bm
    return pl.pallas_call(
        _colsum_body,
        grid=(grid,),
        in_specs=[
            pl.BlockSpec((bm, F), lambda i: (i, 0)),
            pl.BlockSpec((F, F), lambda i: (0, 0)),
            pl.BlockSpec((1, F), lambda i: (0, 0)),
        ],
        out_specs=pl.BlockSpec((1, F), lambda i: (0, 0)),
        out_shape=jax.ShapeDtypeStruct((1, F), jnp.float32),
    )(x, w, b)


def _tail_body(su_ref, si_ref, w2_ref, b2_ref, wo_ref, bo_ref, o_ref):
    vec = (su_ref[...] + si_ref[...]) * (1.0 / float(ND))
    v2 = jnp.dot(vec, w2_ref[...], preferred_element_type=jnp.float32)
    v2 = _gelu(v2 + b2_ref[...])
    o = jnp.dot(v2, wo_ref[...], preferred_element_type=jnp.float32)
    o = o + bo_ref[...]
    o = jnp.where(jnp.isnan(o), 0.0, o)
    o_ref[...] = jnp.clip(o, -_F32_MAX, _F32_MAX)


def _tail(su, si, w2, b2, wo, bo):
    oc = bo.shape[-1]
    return pl.pallas_call(
        _tail_body,
        out_shape=jax.ShapeDtypeStruct((1, oc), jnp.float32),
    )(su, si, w2, b2, wo, bo)


def _fold_conv_weights(p):
    folded = {}
    for ei, nt in enumerate(("user", "item")):
        wkqv = p["w_kqv_" + nt]
        bkqv = p["b_kqv_" + nt]
        Wk, Wq, Wv = wkqv[:, 0:F], wkqv[:, F:2 * F], wkqv[:, 2 * F:3 * F]
        bk, bq, bv = bkqv[0:F], bkqv[F:2 * F], bkqv[2 * F:3 * F]
        kb, vb = [], []
        for h in range(H):
            t = h * 2 + ei
            scale = p["p_rel"][ei][h] / _SQRT_D
            kb.append(p["w_k_rel"][t] * scale)
            vb.append(p["w_v_rel"][t])
        BDk = jax.scipy.linalg.block_diag(*kb)
        BDv = jax.scipy.linalg.block_diag(*vb)
        W = jnp.concatenate([Wq, Wk @ BDk, Wv @ BDv], axis=1)
        b = jnp.concatenate([bq, bk @ BDk, bv @ BDv])
        folded[nt] = (W, b.reshape(1, 3 * F))
    return folded


def _conv(x_user, x_item, p, dstp, srcp, mode, norm_g, norm_b):
    folded = _fold_conv_weights(p)
    qu, ku, vu = _proj(x_user, *folded["user"])
    qi, ki, vi = _proj(x_item, *folded["item"])
    q_cat = jnp.concatenate([qu, qi], axis=0)
    kv_cat = jnp.concatenate(
        [jnp.concatenate([ku, vu], axis=1), jnp.concatenate([ki, vi], axis=1)],
        axis=0)
    Q, KV = _sc_gather(q_cat, kv_cat, dstp, srcp)
    M = _edge_math(Q, KV)
    P = _sc_scatter(M, dstp)
    outs = []
    for i, nt in enumerate(("user", "item")):
        P_nt = lax.slice(P, (0, 0, i * N, 0), (NC, H + 1, (i + 1) * N, D))
        x_nt = x_user if nt == "user" else x_item
        g = norm_g[nt] if norm_g is not None else jnp.ones((1, F), jnp.float32)
        bn = norm_b[nt] if norm_b is not None else jnp.zeros((1, F), jnp.float32)
        outs.append(_post(P_nt, x_nt, p["w_out_" + nt],
                          p["b_out_" + nt].reshape(1, F),
                          p["skip_" + nt].reshape(1, 1), g, bn, mode))
    return outs


def kernel(x_user, x_item, params, edge_index_user_item, edge_index_item_user):
    ei_ui = edge_index_user_item
    ei_iu = edge_index_item_user
    src = jnp.concatenate([ei_ui[0], ei_iu[0] + N])
    dst = jnp.concatenate([ei_ui[1] + N, ei_iu[1]])
    srcp = jnp.zeros((EP,), jnp.int32).at[:E].set(src)
    dstp = jnp.zeros((EP,), jnp.int32).at[:E].set(dst)

    norm_g = {nt: params["norm_g_" + nt].reshape(1, F) for nt in ("user", "item")}
    norm_b = {nt: params["norm_b_" + nt].reshape(1, F) for nt in ("user", "item")}

    hu, hi = _conv(x_user, x_item, params["conv1"], dstp, srcp,
                   "ln_gelu", norm_g, norm_b)
    hu, hi = _conv(hu, hi, params["conv2"], dstp, srcp, "gelu", None, None)
    hu, hi = _conv(hu, hi, params["conv3"], dstp, srcp, "gelu", None, None)

    su = _colsum(hu, params["w_agg1"][0], params["b_agg1"][0].reshape(1, F))
    si = _colsum(hi, params["w_agg1"][1], params["b_agg1"][1].reshape(1, F))
    out = _tail(su, si, params["w_agg2"], params["b_agg2"].reshape(1, -1),
                params["w_out"], params["b_out"].reshape(1, -1))
    return out.reshape(-1)


# speedup vs baseline: 1.1242x; 1.1242x over previous
import functools

import jax
import jax.numpy as jnp
from jax import lax
from jax.experimental import pallas as pl
from jax.experimental.pallas import tpu as pltpu
from jax.experimental.pallas import tpu_sc as plsc

N = 25000
ND = 2 * N
E = 500000
EP = 524288
F = 128
H = 4
D = 32
NC = 2
NS = 16
NW = NC * NS
ACC = 50176

_SQRT_D = 5.656854249492381
_F32_MAX = 3.4028235e38


def _gelu(x):
    return x * 0.5 * (1.0 + lax.erf(x * 0.7071067811865476))


def _proj_body(x_ref, w_ref, b_ref, q_ref, k_ref, v_ref):
    y = jnp.dot(x_ref[...], w_ref[...], preferred_element_type=jnp.float32)
    y = y + b_ref[...]
    q_ref[...] = y[:, 0:F]
    k_ref[...] = y[:, F:2 * F]
    v_ref[...] = y[:, 2 * F:3 * F]


def _proj(x, w, b):
    bm = 1000
    grid = N // bm
    return pl.pallas_call(
        _proj_body,
        grid=(grid,),
        in_specs=[
            pl.BlockSpec((bm, F), lambda i: (i, 0)),
            pl.BlockSpec((F, 3 * F), lambda i: (0, 0)),
            pl.BlockSpec((1, 3 * F), lambda i: (0, 0)),
        ],
        out_specs=[pl.BlockSpec((bm, F), lambda i: (i, 0))] * 3,
        out_shape=[jax.ShapeDtypeStruct((N, F), jnp.float32)] * 3,
    )(x, w, b)


_EPW = EP // NW
_GC = 128
_NB = 2
_GN = _EPW // (_GC * _NB)


def _sc_gather_body(qt, kt, vt, dstp, srcp, qo, ko, vo,
                    idxd, idxs, qr, kr, vr, semg, semw):
    c = lax.axis_index("c")
    s = lax.axis_index("s")
    wid = s * NC + c
    base = wid * _EPW

    def body(g, carry):
        handles = []
        for b in range(_NB):
            off = base + (g * _NB + b) * _GC
            pltpu.sync_copy(dstp.at[pl.ds(off, _GC)], idxd[b])
            pltpu.sync_copy(srcp.at[pl.ds(off, _GC)], idxs[b])

            @pl.when(g > 0)
            def _():
                pltpu.make_async_copy(qr[b], qo.at[pl.ds(off, _GC)],
                                      semw[3 * b]).wait()
                pltpu.make_async_copy(kr[b], ko.at[pl.ds(off, _GC)],
                                      semw[3 * b + 1]).wait()
                pltpu.make_async_copy(vr[b], vo.at[pl.ds(off, _GC)],
                                      semw[3 * b + 2]).wait()

            handles.append((
                pltpu.async_copy(qt.at[idxd[b]], qr[b], semg[3 * b]),
                pltpu.async_copy(kt.at[idxs[b]], kr[b], semg[3 * b + 1]),
                pltpu.async_copy(vt.at[idxs[b]], vr[b], semg[3 * b + 2]),
            ))
        for b in range(_NB):
            off = base + (g * _NB + b) * _GC
            hq, hk, hv = handles[b]
            hq.wait()
            pltpu.async_copy(qr[b], qo.at[pl.ds(off, _GC)], semw[3 * b])
            hk.wait()
            pltpu.async_copy(kr[b], ko.at[pl.ds(off, _GC)], semw[3 * b + 1])
            hv.wait()
            pltpu.async_copy(vr[b], vo.at[pl.ds(off, _GC)], semw[3 * b + 2])
        return carry

    lax.fori_loop(0, _GN, body, 0)
    for b in range(_NB):
        pltpu.make_async_copy(qr[b], qo.at[pl.ds(0, _GC)], semw[3 * b]).wait()
        pltpu.make_async_copy(kr[b], ko.at[pl.ds(0, _GC)], semw[3 * b + 1]).wait()
        pltpu.make_async_copy(vr[b], vo.at[pl.ds(0, _GC)], semw[3 * b + 2]).wait()


def _sc_gather(q_cat, k_cat, v_cat, dstp, srcp):
    f = pl.kernel(
        _sc_gather_body,
        mesh=plsc.VectorSubcoreMesh(core_axis_name="c", subcore_axis_name="s"),
        out_type=[jax.ShapeDtypeStruct((EP, F), jnp.float32)] * 3,
        scratch_types=[
            [pltpu.VMEM((_GC,), jnp.int32) for _ in range(_NB)],
            [pltpu.VMEM((_GC,), jnp.int32) for _ in range(_NB)],
            [pltpu.VMEM((_GC, F), jnp.float32) for _ in range(_NB)],
            [pltpu.VMEM((_GC, F), jnp.float32) for _ in range(_NB)],
            [pltpu.VMEM((_GC, F), jnp.float32) for _ in range(_NB)],
            [pltpu.SemaphoreType.DMA for _ in range(3 * _NB)],
            [pltpu.SemaphoreType.DMA for _ in range(3 * _NB)],
        ],
        compiler_params=pltpu.CompilerParams(use_tc_tiling_on_sc=False),
    )
    return f(q_cat, k_cat, v_cat, dstp, srcp)


_BE = 2048


def _edge_body(q_ref, k_ref, v_ref, m_ref):
    i = pl.program_id(0)
    q = q_ref[...]
    k = k_ref[...]
    v = v_ref[...]
    rows = i * _BE + lax.broadcasted_iota(jnp.int32, (_BE, 1), 0)
    valid = rows < E
    es = []
    for h in range(H):
        sl = slice(h * D, (h + 1) * D)
        s = jnp.sum(q[:, sl] * k[:, sl], axis=1, keepdims=True)
        e = jnp.where(valid, jnp.exp(s), 0.0)
        es.append(e)
        m_ref[h] = v[:, sl] * e
    m_ref[H] = jnp.concatenate(es + [jnp.zeros((_BE, D - H), jnp.float32)], axis=1)


def _edge_math(Q, K, V):
    grid = EP // _BE
    return pl.pallas_call(
        _edge_body,
        grid=(grid,),
        in_specs=[pl.BlockSpec((_BE, F), lambda i: (i, 0))] * 3,
        out_specs=pl.BlockSpec((H + 1, _BE, D), lambda i: (0, i, 0)),
        out_shape=jax.ShapeDtypeStruct((H + 1, EP, D), jnp.float32),
    )(Q, K, V)


_EPC = EP // NC
_EPT = _EPC // NS
_SCK = 512
_SN = _EPT // _SCK
_ZR = ACC // NS


def _sc_scatter_body(m, dstp, po, vals, idx, acc):
    c = lax.axis_index("c")
    s = lax.axis_index("s")

    for h in range(H + 1):
        def zb(i, carry):
            vals[i, pl.ds(0, 16)] = jnp.zeros((16,), jnp.float32)
            vals[i, pl.ds(16, 16)] = jnp.zeros((16,), jnp.float32)
            return carry

        lax.fori_loop(0, _SCK, zb, 0)
        for j in range(6):
            pltpu.sync_copy(vals, acc.at[pl.ds(s * _ZR + j * _SCK, _SCK)])
        pltpu.sync_copy(vals.at[pl.ds(0, 64)],
                        acc.at[pl.ds(s * _ZR + 6 * _SCK, 64)])
        plsc.subcore_barrier()

        base = c * _EPC + s * _EPT

        def body(g, carry):
            off = base + g * _SCK
            pltpu.sync_copy(dstp.at[pl.ds(off, _SCK)], idx)
            pltpu.sync_copy(m.at[h, pl.ds(off, _SCK)], vals)
            pltpu.sync_copy(vals, acc.at[idx], add=True)
            return carry

        lax.fori_loop(0, _SN, body, 0)
        plsc.subcore_barrier()

        for j in range(6):
            r0 = s * _ZR + j * _SCK
            pltpu.sync_copy(acc.at[pl.ds(r0, _SCK)], po.at[c, h, pl.ds(r0, _SCK)])
        r0 = s * _ZR + 6 * _SCK
        pltpu.sync_copy(acc.at[pl.ds(r0, 64)], po.at[c, h, pl.ds(r0, 64)])
        plsc.subcore_barrier()


def _sc_scatter(M, dstp):
    f = pl.kernel(
        _sc_scatter_body,
        mesh=plsc.VectorSubcoreMesh(core_axis_name="c", subcore_axis_name="s"),
        out_type=jax.ShapeDtypeStruct((NC, H + 1, ACC, D), jnp.float32),
        scratch_types=[
            pltpu.VMEM((_SCK, D), jnp.float32),
            pltpu.VMEM((_SCK,), jnp.int32),
            pltpu.VMEM_SHARED((ACC, D), jnp.float32),
        ],
        compiler_params=pltpu.CompilerParams(use_tc_tiling_on_sc=False),
    )
    return f(M, dstp)


def _post_body(p_ref, x_ref, w_ref, b_ref, sk_ref, g_ref, bn_ref, o_ref,
               *, mode):
    p = p_ref[...]
    ps = p[0] + p[1]
    cols = []
    for h in range(H):
        cols.append(ps[h] / (ps[H][:, h:h + 1] + 1e-16))
    agg = jnp.concatenate(cols, axis=1)
    a = jnp.dot(_gelu(agg), w_ref[...], preferred_element_type=jnp.float32)
    a = a + b_ref[...]
    sk = jax.nn.sigmoid(sk_ref[0, 0])
    a = sk * a + (1.0 - sk) * x_ref[...]
    if mode == "ln_gelu":
        mu = jnp.mean(a, axis=-1, keepdims=True)
        var = jnp.mean((a - mu) * (a - mu), axis=-1, keepdims=True)
        a = (a - mu) / jnp.sqrt(var + 1e-5) * g_ref[...] + bn_ref[...]
    o_ref[...] = _gelu(a)


def _post(P_nt, x_nt, w_out, b_out, skip, g, bn, mode):
    bm = 1000
    grid = N // bm
    body = functools.partial(_post_body, mode=mode)
    return pl.pallas_call(
        body,
        grid=(grid,),
        in_specs=[
            pl.BlockSpec((NC, H + 1, bm, D), lambda i: (0, 0, i, 0)),
            pl.BlockSpec((bm, F), lambda i: (i, 0)),
            pl.BlockSpec((F, F), lambda i: (0, 0)),
            pl.BlockSpec((1, F), lambda i: (0, 0)),
            pl.BlockSpec((1, 1), lambda i: (0, 0)),
            pl.BlockSpec((1, F), lambda i: (0, 0)),
            pl.BlockSpec((1, F), lambda i: (0, 0)),
        ],
        out_specs=pl.BlockSpec((bm, F), lambda i: (i, 0)),
        out_shape=jax.ShapeDtypeStruct((N, F), jnp.float32),
    )(P_nt, x_nt, w_out, b_out, skip, g, bn)


def _colsum_body(x_ref, w_ref, b_ref, o_ref):
    i = pl.program_id(0)
    a = jnp.dot(x_ref[...], w_ref[...], preferred_element_type=jnp.float32)
    a = _gelu(a + b_ref[...])
    s = jnp.sum(a, axis=0, keepdims=True)

    @pl.when(i == 0)
    def _():
        o_ref[...] = s

    @pl.when(i > 0)
    def _():
        o_ref[...] += s


def _colsum(x, w, b):
    bm = 1000
    grid = N // bm
    return pl.pallas_call(
        _colsum_body,
        grid=(grid,),
        in_specs=[
            pl.BlockSpec((bm, F), lambda i: (i, 0)),
            pl.BlockSpec((F, F), lambda i: (0, 0)),
            pl.BlockSpec((1, F), lambda i: (0, 0)),
        ],
        out_specs=pl.BlockSpec((1, F), lambda i: (0, 0)),
        out_shape=jax.ShapeDtypeStruct((1, F), jnp.float32),
    )(x, w, b)


def _tail_body(su_ref, si_ref, w2_ref, b2_ref, wo_ref, bo_ref, o_ref):
    vec = (su_ref[...] + si_ref[...]) * (1.0 / float(ND))
    v2 = jnp.dot(vec, w2_ref[...], preferred_element_type=jnp.float32)
    v2 = _gelu(v2 + b2_ref[...])
    o = jnp.dot(v2, wo_ref[...], preferred_element_type=jnp.float32)
    o = o + bo_ref[...]
    o = jnp.where(jnp.isnan(o), 0.0, o)
    o_ref[...] = jnp.clip(o, -_F32_MAX, _F32_MAX)


def _tail(su, si, w2, b2, wo, bo):
    oc = bo.shape[-1]
    return pl.pallas_call(
        _tail_body,
        out_shape=jax.ShapeDtypeStruct((1, oc), jnp.float32),
    )(su, si, w2, b2, wo, bo)


def _fold_conv_weights(p):
    folded = {}
    for ei, nt in enumerate(("user", "item")):
        wkqv = p["w_kqv_" + nt]
        bkqv = p["b_kqv_" + nt]
        Wk, Wq, Wv = wkqv[:, 0:F], wkqv[:, F:2 * F], wkqv[:, 2 * F:3 * F]
        bk, bq, bv = bkqv[0:F], bkqv[F:2 * F], bkqv[2 * F:3 * F]
        kb, vb = [], []
        for h in range(H):
            t = h * 2 + ei
            scale = p["p_rel"][ei][h] / _SQRT_D
            kb.append(p["w_k_rel"][t] * scale)
            vb.append(p["w_v_rel"][t])
        BDk = jax.scipy.linalg.block_diag(*kb)
        BDv = jax.scipy.linalg.block_diag(*vb)
        W = jnp.concatenate([Wq, Wk @ BDk, Wv @ BDv], axis=1)
        b = jnp.concatenate([bq, bk @ BDk, bv @ BDv])
        folded[nt] = (W, b.reshape(1, 3 * F))
    return folded


def _conv(x_user, x_item, p, dstp, srcp, mode, norm_g, norm_b):
    folded = _fold_conv_weights(p)
    qu, ku, vu = _proj(x_user, *folded["user"])
    qi, ki, vi = _proj(x_item, *folded["item"])
    q_cat = jnp.concatenate([qu, qi], axis=0)
    k_cat = jnp.concatenate([ku, ki], axis=0)
    v_cat = jnp.concatenate([vu, vi], axis=0)
    Q, K, V = _sc_gather(q_cat, k_cat, v_cat, dstp, srcp)
    M = _edge_math(Q, K, V)
    P = _sc_scatter(M, dstp)
    outs = []
    for i, nt in enumerate(("user", "item")):
        P_nt = lax.slice(P, (0, 0, i * N, 0), (NC, H + 1, (i + 1) * N, D))
        x_nt = x_user if nt == "user" else x_item
        g = norm_g[nt] if norm_g is not None else jnp.ones((1, F), jnp.float32)
        bn = norm_b[nt] if norm_b is not None else jnp.zeros((1, F), jnp.float32)
        outs.append(_post(P_nt, x_nt, p["w_out_" + nt],
                          p["b_out_" + nt].reshape(1, F),
                          p["skip_" + nt].reshape(1, 1), g, bn, mode))
    return outs


def kernel(x_user, x_item, params, edge_index_user_item, edge_index_item_user):
    ei_ui = edge_index_user_item
    ei_iu = edge_index_item_user
    src = jnp.concatenate([ei_ui[0], ei_iu[0] + N])
    dst = jnp.concatenate([ei_ui[1] + N, ei_iu[1]])
    srcp = jnp.zeros((EP,), jnp.int32).at[:E].set(src)
    dstp = jnp.zeros((EP,), jnp.int32).at[:E].set(dst)

    norm_g = {nt: params["norm_g_" + nt].reshape(1, F) for nt in ("user", "item")}
    norm_b = {nt: params["norm_b_" + nt].reshape(1, F) for nt in ("user", "item")}

    hu, hi = _conv(x_user, x_item, params["conv1"], dstp, srcp,
                   "ln_gelu", norm_g, norm_b)
    hu, hi = _conv(hu, hi, params["conv2"], dstp, srcp, "gelu", None, None)
    hu, hi = _conv(hu, hi, params["conv3"], dstp, srcp, "gelu", None, None)

    su = _colsum(hu, params["w_agg1"][0], params["b_agg1"][0].reshape(1, F))
    si = _colsum(hi, params["w_agg1"][1], params["b_agg1"][1].reshape(1, F))
    out = _tail(su, si, params["w_agg2"], params["b_agg2"].reshape(1, -1),
                params["w_out"], params["b_out"].reshape(1, -1))
    return out.reshape(-1)
